# baseline (speedup 1.0000x reference)
import functools

import jax
import jax.numpy as jnp
from jax import lax
from jax.experimental import pallas as pl
from jax.experimental.pallas import tpu as pltpu
from jax.experimental.pallas import tpu_sc as plsc

TIME_CLIP = 2048
HIDDEN = 32
BATCH = 1024
HIST = 50
PAIRS = HIST * HIST
LANES = 16
PAIRS_PAD = 2512
NCHUNK = PAIRS_PAD // LANES


def kernel(timestamps, time_emb):
    info = plsc.get_sparse_core_info()
    num_workers = info.num_cores * info.num_subcores
    b_per_w = BATCH // num_workers

    flat = jnp.arange(PAIRS_PAD, dtype=jnp.int32)
    valid = flat < PAIRS
    i_idx = jnp.where(valid, flat // HIST, 0).astype(jnp.int32)
    j_idx = jnp.where(valid, flat % HIST, 0).astype(jnp.int32)

    mesh = plsc.VectorSubcoreMesh(core_axis_name="c", subcore_axis_name="s")

    @functools.partial(
        pl.kernel,
        mesh=mesh,
        compiler_params=pltpu.CompilerParams(
            needs_layout_passes=False, use_tc_tiling_on_sc=False),
        out_type=jax.ShapeDtypeStruct((BATCH, PAIRS, HIDDEN), jnp.float32),
        scratch_types=[
            pltpu.VMEM((b_per_w * HIST,), jnp.int32),
            pltpu.VMEM((PAIRS_PAD,), jnp.int32),
            pltpu.VMEM((PAIRS_PAD,), jnp.int32),
            pltpu.VMEM((PAIRS_PAD,), jnp.int32),
            pltpu.VMEM((PAIRS_PAD, HIDDEN), jnp.float32),
            pltpu.SemaphoreType.DMA,
        ],
    )
    def sc_kernel(ts_hbm, tab_hbm, ii_hbm, jj_hbm, out_hbm,
                  ts_v, ii_v, jj_v, idx_v, rows_v, sem):
        wid = lax.axis_index("s") * info.num_cores + lax.axis_index("c")
        b0 = wid * b_per_w

        pltpu.sync_copy(ts_hbm.at[pl.ds(b0 * HIST, b_per_w * HIST)], ts_v)
        pltpu.sync_copy(ii_hbm, ii_v)
        pltpu.sync_copy(jj_hbm, jj_v)

        def batch_body(b, carry):
            boff = b * HIST

            def chunk_body(c, carry2):
                off = c * LANES
                ii = ii_v[pl.ds(off, LANES)]
                jj = jj_v[pl.ds(off, LANES)]
                ti = plsc.load_gather(ts_v, [ii + boff])
                tj = plsc.load_gather(ts_v, [jj + boff])
                d = jnp.minimum(jnp.abs(ti - tj), TIME_CLIP)
                idx_v[pl.ds(off, LANES)] = d
                return carry2

            lax.fori_loop(0, NCHUNK, chunk_body, 0, unroll=4)

            pltpu.async_copy(tab_hbm.at[idx_v], rows_v, sem).wait()
            pltpu.sync_copy(rows_v.at[pl.ds(0, PAIRS)], out_hbm.at[b0 + b])
            return carry

        lax.fori_loop(0, b_per_w, batch_body, 0)

    out = sc_kernel(timestamps.reshape(BATCH * HIST), time_emb, i_idx, j_idx)
    return out.reshape(BATCH, HIST, HIST, HIDDEN)


# speedup vs baseline: 10.4508x; 10.4508x over previous
import functools

import jax
import jax.numpy as jnp
from jax import lax
from jax.experimental import pallas as pl
from jax.experimental.pallas import tpu as pltpu
from jax.experimental.pallas import tpu_sc as plsc

TIME_CLIP = 2048
HIDDEN = 32
BATCH = 1024
HIST = 50
PAIRS = HIST * HIST
LANES = 16
PAIRS_PAD = 2528
NCHUNK = PAIRS_PAD // LANES
H0 = 1264
H1 = PAIRS - H0
TAB_ROWS = TIME_CLIP + 1


def kernel(timestamps, time_emb):
    info = plsc.get_sparse_core_info()
    num_workers = info.num_cores * info.num_subcores
    b_per_w = BATCH // num_workers

    flat = jnp.arange(PAIRS_PAD, dtype=jnp.int32)
    valid = flat < PAIRS
    i_idx = jnp.where(valid, flat // HIST, 0).astype(jnp.int32)
    j_idx = jnp.where(valid, flat % HIST, 0).astype(jnp.int32)

    mesh = plsc.VectorSubcoreMesh(core_axis_name="c", subcore_axis_name="s")

    @functools.partial(
        pl.kernel,
        mesh=mesh,
        compiler_params=pltpu.CompilerParams(
            needs_layout_passes=False, use_tc_tiling_on_sc=False),
        out_type=jax.ShapeDtypeStruct((BATCH, PAIRS, HIDDEN), jnp.float32),
        scratch_types=[
            pltpu.VMEM((b_per_w * HIST,), jnp.int32),
            pltpu.VMEM((PAIRS_PAD,), jnp.int32),
            pltpu.VMEM((PAIRS_PAD,), jnp.int32),
            pltpu.VMEM((PAIRS_PAD,), jnp.int32),
            pltpu.VMEM((PAIRS_PAD,), jnp.int32),
            pltpu.VMEM((H0, HIDDEN), jnp.float32),
            pltpu.VMEM((H1, HIDDEN), jnp.float32),
            pltpu.VMEM_SHARED((TAB_ROWS, HIDDEN), jnp.float32),
            pltpu.SemaphoreType.DMA,
            pltpu.SemaphoreType.DMA,
            pltpu.SemaphoreType.DMA,
            pltpu.SemaphoreType.DMA,
        ],
    )
    def sc_kernel(ts_hbm, tab_hbm, ii_hbm, jj_hbm, out_hbm,
                  ts_v, ii_v, jj_v, ib0, ib1, rb0, rb1, tab_sh,
                  gs0, gs1, ss0, ss1):
        cid = lax.axis_index("c")
        sid = lax.axis_index("s")
        wid = sid * info.num_cores + cid
        b0 = wid * b_per_w
        nb = b_per_w

        pltpu.sync_copy(ts_hbm.at[pl.ds(b0 * HIST, b_per_w * HIST)], ts_v)
        pltpu.sync_copy(ii_hbm, ii_v)
        pltpu.sync_copy(jj_hbm, jj_v)

        trow = sid * 128
        pltpu.sync_copy(tab_hbm.at[pl.ds(trow, 128)], tab_sh.at[pl.ds(trow, 128)])

        @pl.when(sid == info.num_subcores - 1)
        def _():
            pltpu.sync_copy(tab_hbm.at[pl.ds(TAB_ROWS - 1, 1)],
                            tab_sh.at[pl.ds(TAB_ROWS - 1, 1)])

        plsc.subcore_barrier()

        def compute_idx(b, ib_ref):
            boff = b * HIST

            def chunk_body(c, carry):
                off = c * LANES
                ii = ii_v[pl.ds(off, LANES)]
                jj = jj_v[pl.ds(off, LANES)]
                ti = plsc.load_gather(ts_v, [ii + boff])
                tj = plsc.load_gather(ts_v, [jj + boff])
                d = jnp.minimum(jnp.abs(ti - tj), TIME_CLIP)
                ib_ref[pl.ds(off, LANES)] = d
                return carry

            lax.fori_loop(0, NCHUNK, chunk_body, 0, unroll=4)

        def gather0(ib_ref):
            return pltpu.make_async_copy(
                tab_sh.at[ib_ref.at[pl.ds(0, H0)]], rb0, gs0)

        def gather1(ib_ref):
            return pltpu.make_async_copy(
                tab_sh.at[ib_ref.at[pl.ds(H0, H1)]], rb1, gs1)

        def scatter0(b):
            return pltpu.make_async_copy(
                rb0, out_hbm.at[b0 + b, pl.ds(0, H0)], ss0)

        def scatter1(b):
            return pltpu.make_async_copy(
                rb1, out_hbm.at[b0 + b, pl.ds(H0, H1)], ss1)

        compute_idx(0, ib0)
        gather0(ib0).start()
        gather1(ib0).start()

        def pair_body(k, carry):
            e = 2 * k
            o = 2 * k + 1
            nxt = lax.rem(o + 1, nb)

            compute_idx(o, ib1)
            gather0(ib0).wait()
            scatter0(e).start()
            gather1(ib0).wait()
            scatter1(e).start()
            scatter0(e).wait()
            gather0(ib1).start()
            scatter1(e).wait()
            gather1(ib1).start()

            compute_idx(nxt, ib0)
            gather0(ib1).wait()
            scatter0(o).start()
            gather1(ib1).wait()
            scatter1(o).start()
            scatter0(o).wait()
            gather0(ib0).start()
            scatter1(o).wait()
            gather1(ib0).start()
            return carry

        lax.fori_loop(0, nb // 2, pair_body, 0)
        gather0(ib0).wait()
        gather1(ib0).wait()

    out = sc_kernel(timestamps.reshape(BATCH * HIST), time_emb, i_idx, j_idx)
    return out.reshape(BATCH, HIST, HIST, HIDDEN)
